# Initial kernel scaffold; baseline (speedup 1.0000x reference)
import jax
import jax.numpy as jnp
from jax.experimental import pallas as pl


def kernel(x, Wg, bg, W1, b1, W2, b2):
    raise NotImplementedError("write your pallas kernel here")


# speedup vs baseline: 2.0908x; 2.0908x over previous
import functools

import jax
import jax.numpy as jnp
from jax.experimental import pallas as pl
from jax.experimental.pallas import tpu as pltpu

E = 8
K = 2
D_IN = 768
D_HID = 1024
D_OUT = 768
N = 8192
A = N * K
BLK = 256
S = A + E * BLK
NB = S // BLK


def _ffn_block(be_ref, xs_ref, w_ref, W1_ref, b1_ref, W2_ref, ys_ref):
    xb = xs_ref[...]
    h = jnp.dot(xb, W1_ref[0], preferred_element_type=jnp.float32)
    h = h + b1_ref[0]
    h = 0.5 * h * (1.0 + jax.lax.erf(h * 0.7071067811865476))
    h = h * w_ref[...]
    ys_ref[...] = jnp.dot(h, W2_ref[0], preferred_element_type=jnp.float32)


@functools.partial(jax.jit, static_argnames=("interpret",))
def _grouped_ffn(xs, ws, block_expert, W1, b1, W2, interpret=False):
    grid_spec = pltpu.PrefetchScalarGridSpec(
        num_scalar_prefetch=1,
        grid=(NB,),
        in_specs=[
            pl.BlockSpec((BLK, D_IN), lambda i, be: (i, 0)),
            pl.BlockSpec((BLK, 1), lambda i, be: (i, 0)),
            pl.BlockSpec((1, D_IN, D_HID), lambda i, be: (be[i], 0, 0)),
            pl.BlockSpec((1, 1, D_HID), lambda i, be: (be[i], 0, 0)),
            pl.BlockSpec((1, D_HID, D_OUT), lambda i, be: (be[i], 0, 0)),
        ],
        out_specs=pl.BlockSpec((BLK, D_OUT), lambda i, be: (i, 0)),
    )
    return pl.pallas_call(
        _ffn_block,
        grid_spec=grid_spec,
        out_shape=jax.ShapeDtypeStruct((S, D_OUT), jnp.float32),
        interpret=interpret,
    )(block_expert, xs, ws, W1, b1[:, None, :], W2)


def kernel(x, Wg, bg, W1, b1, W2, b2, interpret=False):
    logits = x @ Wg + bg
    i1 = jnp.argmax(logits, axis=-1).astype(jnp.int32)
    l1 = jnp.max(logits, axis=-1)
    masked = jnp.where(jax.nn.one_hot(i1, E, dtype=jnp.bool_), -jnp.inf, logits)
    i2 = jnp.argmax(masked, axis=-1).astype(jnp.int32)
    l2 = jnp.max(masked, axis=-1)
    p1 = jax.nn.sigmoid(l1 - l2)
    p2 = 1.0 - p1

    probs = jax.nn.softmax(logits, axis=-1)
    eu = jnp.mean(probs, axis=0)
    uniform = jnp.float32(1.0 / E)
    aux_loss = jnp.sum(eu * jnp.log(uniform) - jnp.log(eu) * uniform)

    e_flat = jnp.stack([i1, i2], axis=1).reshape(-1)
    p_flat = jnp.stack([p1, p2], axis=1).reshape(-1)
    oh = jax.nn.one_hot(e_flat, E, dtype=jnp.int32)
    cum = jnp.cumsum(oh, axis=0)
    counts = cum[-1]
    rank = jnp.take_along_axis(cum, e_flat[:, None], axis=1)[:, 0] - 1
    padded = ((counts + BLK - 1) // BLK) * BLK
    ends = jnp.cumsum(padded)
    starts = ends - padded
    pos = starts[e_flat] + rank
    tok = (jnp.arange(A, dtype=jnp.int32) // K)
    tok_for_pos = jnp.zeros((S,), jnp.int32).at[pos].set(tok, mode="drop")
    w_for_pos = jnp.zeros((S,), jnp.float32).at[pos].set(p_flat, mode="drop")
    block_expert = jnp.clip(
        jnp.searchsorted(ends, jnp.arange(NB, dtype=jnp.int32) * BLK, side="right"),
        0, E - 1).astype(jnp.int32)

    xs = jnp.take(x, tok_for_pos, axis=0, mode="clip")
    ys = _grouped_ffn(xs, w_for_pos[:, None], block_expert, W1, b1, W2,
                      interpret=interpret)
    pos_pairs = pos.reshape(N, K)
    sel = jnp.take(ys, pos_pairs.reshape(-1), axis=0, mode="clip")
    out = sel.reshape(N, K, D_OUT).sum(axis=1)

    p_dense = (p1[:, None] * jax.nn.one_hot(i1, E, dtype=jnp.float32)
               + p2[:, None] * jax.nn.one_hot(i2, E, dtype=jnp.float32))
    out = out + p_dense @ b2
    return (out, aux_loss)
